# baseline (speedup 1.0000x reference)
import functools

import jax
import jax.numpy as jnp
from jax import lax
from jax.experimental import pallas as pl
from jax.experimental.pallas import tpu as pltpu
from jax.experimental.pallas import tpu_sc as plsc

BATCH = 16384
EMBED_DIM = 64
NUM_CORES = 2
NUM_SUBCORES = 16
NUM_WORKERS = NUM_CORES * NUM_SUBCORES
B_PER_W = BATCH // NUM_WORKERS
CHUNK = 128
NCHUNK = B_PER_W // CHUNK


def _gather3(uid2d, mid2d, gid2d, user_table, movie_table, genre_table):
    mesh = plsc.VectorSubcoreMesh(core_axis_name="c", subcore_axis_name="s")
    out = jax.ShapeDtypeStruct((BATCH, EMBED_DIM), jnp.float32)

    @functools.partial(
        pl.kernel,
        mesh=mesh,
        out_type=(out, out, out),
        compiler_params=pltpu.CompilerParams(use_tc_tiling_on_sc=False),
        scratch_types=[
            pltpu.VMEM((NCHUNK, CHUNK), jnp.int32),
            pltpu.VMEM((NCHUNK, CHUNK), jnp.int32),
            pltpu.VMEM((NCHUNK, CHUNK), jnp.int32),
            pltpu.VMEM((B_PER_W, EMBED_DIM), jnp.float32),
            pltpu.VMEM((B_PER_W, EMBED_DIM), jnp.float32),
            pltpu.VMEM((B_PER_W, EMBED_DIM), jnp.float32),
            pltpu.SemaphoreType.DMA,
        ],
    )
    def k(uid_h, mid_h, gid_h, ut_h, mt_h, gt_h, ou_h, om_h, og_h,
          uidx, midx, gidx, urows, mrows, grows, sem):
        wid = lax.axis_index("s") * NUM_CORES + lax.axis_index("c")
        base = wid * B_PER_W
        row0 = wid * NCHUNK
        pltpu.sync_copy(uid_h.at[pl.ds(row0, NCHUNK)], uidx)
        pltpu.sync_copy(mid_h.at[pl.ds(row0, NCHUNK)], midx)
        pltpu.sync_copy(gid_h.at[pl.ds(row0, NCHUNK)], gidx)
        handles = []
        for idx, tbl, rows in (
            (uidx, ut_h, urows),
            (midx, mt_h, mrows),
            (gidx, gt_h, grows),
        ):
            for j in range(NCHUNK):
                handles.append(
                    pltpu.async_copy(
                        tbl.at[idx.at[j]],
                        rows.at[pl.ds(j * CHUNK, CHUNK)],
                        sem,
                    )
                )
        for h in handles:
            h.wait()
        pltpu.sync_copy(urows, ou_h.at[pl.ds(base, B_PER_W)])
        pltpu.sync_copy(mrows, om_h.at[pl.ds(base, B_PER_W)])
        pltpu.sync_copy(grows, og_h.at[pl.ds(base, B_PER_W)])

    return k(uid2d, mid2d, gid2d, user_table, movie_table, genre_table)


@jax.jit
def kernel(uid, mid, gid, user_table, movie_table, genre_table):
    uid2 = uid.astype(jnp.int32).reshape(NUM_WORKERS * NCHUNK, CHUNK)
    mid2 = mid.astype(jnp.int32).reshape(NUM_WORKERS * NCHUNK, CHUNK)
    gid2 = gid.astype(jnp.int32).reshape(NUM_WORKERS * NCHUNK, CHUNK)
    return _gather3(uid2, mid2, gid2, user_table, movie_table, genre_table)


# speedup vs baseline: 1.6775x; 1.6775x over previous
import functools

import jax
import jax.numpy as jnp
from jax import lax
from jax.experimental import pallas as pl
from jax.experimental.pallas import tpu as pltpu
from jax.experimental.pallas import tpu_sc as plsc

BATCH = 16384
EMBED_DIM = 64
NUM_CORES = 2
NUM_SUBCORES = 16
NUM_WORKERS = NUM_CORES * NUM_SUBCORES
B_PER_W = BATCH // NUM_WORKERS
CHUNK = 64
NCHUNK = B_PER_W // CHUNK


def _gather3(uid2, mid2, gid2, ut3, mt3, gt3):
    mesh = plsc.VectorSubcoreMesh(core_axis_name="c", subcore_axis_name="s")
    out = jax.ShapeDtypeStruct((NUM_WORKERS * NCHUNK, CHUNK, EMBED_DIM),
                               jnp.float32)

    @functools.partial(
        pl.kernel,
        mesh=mesh,
        out_type=(out, out, out),
        scratch_types=[
            pltpu.VMEM((B_PER_W + 16,), jnp.int32),
            pltpu.VMEM((CHUNK, 8, EMBED_DIM), jnp.float32),
            pltpu.VMEM((CHUNK, EMBED_DIM), jnp.float32),
            pltpu.SemaphoreType.DMA,
        ],
    )
    def k(uid_h, mid_h, gid_h, ut_h, mt_h, gt_h, ou_h, om_h, og_h,
          idxv, chunk, ebuf, sem):
        wid = lax.axis_index("s") * NUM_CORES + lax.axis_index("c")
        for idx_h, tbl_h, out_h in ((uid_h, ut_h, ou_h), (mid_h, mt_h, om_h),
                                    (gid_h, gt_h, og_h)):
            pltpu.sync_copy(idx_h.at[wid], idxv.at[pl.ds(0, B_PER_W)])

            def body(g, carry):
                def fire(i, c2):
                    ii = idxv[pl.ds(g * CHUNK + i, 16)][0]
                    pltpu.async_copy(tbl_h.at[ii >> 3], chunk.at[i], sem)
                    return c2

                lax.fori_loop(0, CHUNK, fire, 0)
                pltpu.make_async_copy(tbl_h.at[pl.ds(0, CHUNK)], chunk,
                                      sem).wait()

                def extract(i, c2):
                    s = idxv[pl.ds(g * CHUNK + i, 16)][0] & 7
                    for c in range(EMBED_DIM // 16):
                        ebuf[i, pl.ds(c * 16, 16)] = chunk[i, s,
                                                           pl.ds(c * 16, 16)]
                    return c2

                lax.fori_loop(0, CHUNK, extract, 0)
                pltpu.sync_copy(ebuf, out_h.at[wid * NCHUNK + g])
                return carry

            lax.fori_loop(0, NCHUNK, body, 0)

    return k(uid2, mid2, gid2, ut3, mt3, gt3)


@jax.jit
def kernel(uid, mid, gid, user_table, movie_table, genre_table):
    uid2 = uid.astype(jnp.int32).reshape(NUM_WORKERS, B_PER_W)
    mid2 = mid.astype(jnp.int32).reshape(NUM_WORKERS, B_PER_W)
    gid2 = gid.astype(jnp.int32).reshape(NUM_WORKERS, B_PER_W)
    ut3 = user_table.reshape(-1, 8, EMBED_DIM)
    mt3 = movie_table.reshape(-1, 8, EMBED_DIM)
    gt3 = genre_table.reshape(-1, 8, EMBED_DIM)
    ou, om, og = _gather3(uid2, mid2, gid2, ut3, mt3, gt3)
    return (
        ou.reshape(BATCH, EMBED_DIM),
        om.reshape(BATCH, EMBED_DIM),
        og.reshape(BATCH, EMBED_DIM),
    )
